# baseline (speedup 1.0000x reference)
import jax
import jax.numpy as jnp
from jax import lax
from jax.experimental import pallas as pl
from jax.experimental.pallas import tpu as pltpu
from jax.experimental.pallas import tpu_sc as plsc

F32 = jnp.float32
BF16 = jnp.bfloat16

_N = 10000
_E = 320000
_FEAT = 128
_HID = 256

_TW_G = 288
_TW_E = 256
_CW = 16

_NC, _NS = 2, 16
_NW = _NC * _NS

_GC = 200
_GSUB = 40
_CH = 80
_ACC_R = 10016
_RPT = _ACC_R // _NS


def _sc_gather(table_a, table_b, src, dst):
    per_w = _E // _NW
    n_ch = per_w // _GC
    mesh = plsc.VectorSubcoreMesh(core_axis_name="c", subcore_axis_name="s")

    def body(ta_hbm, tb_hbm, src_hbm, dst_hbm, gs_hbm, gd_hbm,
             idxs_v, idxd_v, rs_v, rd_v, sem):
        c = lax.axis_index("c")
        s = lax.axis_index("s")
        wid = s * _NC + c
        base = wid * per_w

        def step(i, carry):
            off = base + i * _GC
            pltpu.sync_copy(src_hbm.at[pl.ds(off, _GC)], idxs_v)
            pltpu.sync_copy(dst_hbm.at[pl.ds(off, _GC)], idxd_v)
            descs = []
            for j in range(_GC // _GSUB):
                r = pl.ds(j * _GSUB, _GSUB)
                descs.append(pltpu.async_copy(
                    ta_hbm.at[idxs_v.at[r]], rs_v.at[r], sem))
                descs.append(pltpu.async_copy(
                    tb_hbm.at[idxd_v.at[r]], rd_v.at[r], sem))
            for d in descs:
                d.wait()
            pltpu.sync_copy(rs_v, gs_hbm.at[pl.ds(off, _GC)])
            pltpu.sync_copy(rd_v, gd_hbm.at[pl.ds(off, _GC)])
            return carry

        lax.fori_loop(0, n_ch, step, 0)

    f = pl.kernel(
        body,
        out_type=(
            jax.ShapeDtypeStruct((_E, _TW_G), BF16),
            jax.ShapeDtypeStruct((_E, _TW_G), BF16),
        ),
        mesh=mesh,
        scratch_types=[
            pltpu.VMEM((_GC,), jnp.int32),
            pltpu.VMEM((_GC,), jnp.int32),
            pltpu.VMEM((_GC, _TW_G), BF16),
            pltpu.VMEM((_GC, _TW_G), BF16),
            pltpu.SemaphoreType.DMA,
        ],
        compiler_params=pltpu.CompilerParams(use_tc_tiling_on_sc=False),
    )
    return f(table_a, table_b, src, dst)


_BP = 2000


def _node_pre(h2, pos2, w1a, w1b, b1):
    def body(h_ref, pos_ref, w1a_ref, w1b_ref, b1_ref, ta_ref, tb_ref):
        hh = h_ref[...]
        a = jnp.dot(hh, w1a_ref[...], preferred_element_type=F32) + b1_ref[...]
        b = jnp.dot(hh, w1b_ref[...], preferred_element_type=F32)
        pose = pos_ref[...].astype(BF16)
        zpad = jnp.zeros((_BP, _TW_G - _HID - 3), BF16)
        ta_ref[:, :_HID] = a.astype(BF16)
        ta_ref[:, _HID:_HID + 3] = pose
        ta_ref[:, _HID + 3:] = zpad
        tb_ref[:, :_HID] = b.astype(BF16)
        tb_ref[:, _HID:_HID + 3] = pose
        tb_ref[:, _HID + 3:] = zpad

    def wspec(r, c):
        return pl.BlockSpec((r, c), lambda i: (0, 0))

    return pl.pallas_call(
        body,
        grid=(_N // _BP,),
        in_specs=[
            pl.BlockSpec((_BP, _FEAT), lambda i: (i, 0)),
            pl.BlockSpec((_BP, 3), lambda i: (i, 0)),
            wspec(_FEAT, _HID), wspec(_FEAT, _HID), wspec(1, _HID),
        ],
        out_specs=(
            pl.BlockSpec((_BP, _TW_G), lambda i: (i, 0)),
            pl.BlockSpec((_BP, _TW_G), lambda i: (i, 0)),
        ),
        out_shape=(
            jax.ShapeDtypeStruct((_N, _TW_G), BF16),
            jax.ShapeDtypeStruct((_N, _TW_G), BF16),
        ),
    )(h2, pos2, w1a, w1b, b1)


def _sc_scatter(eout, eoutc, dst):
    per_c = _E // _NC
    per_t = per_c // _NS
    n_ch = per_t // _CH
    mesh = plsc.VectorSubcoreMesh(core_axis_name="c", subcore_axis_name="s")

    def body(eout_hbm, eoutc_hbm, dst_hbm, agg_hbm, aggc_hbm,
             dstc_v, rows_v, rowsc_v, acc_sh, accc_sh, sem):
        c = lax.axis_index("c")
        s = lax.axis_index("s")

        zb16 = jnp.zeros((16,), BF16)
        zf16 = jnp.zeros((16,), F32)

        def zrow(i, carry):
            for j in range(_TW_E // 16):
                rows_v[i, pl.ds(j * 16, 16)] = zb16
            rowsc_v[i, pl.ds(0, 16)] = zf16
            return carry

        lax.fori_loop(0, _CH, zrow, 0)
        r0 = s * _RPT
        for k in range(_RPT // _CH):
            pltpu.sync_copy(rows_v, acc_sh.at[pl.ds(r0 + k * _CH, _CH)])
            pltpu.sync_copy(rowsc_v, accc_sh.at[pl.ds(r0 + k * _CH, _CH)])
        rem = _RPT - (_RPT // _CH) * _CH
        pltpu.sync_copy(rows_v.at[pl.ds(0, rem)],
                        acc_sh.at[pl.ds(r0 + (_RPT // _CH) * _CH, rem)])
        pltpu.sync_copy(rowsc_v.at[pl.ds(0, rem)],
                        accc_sh.at[pl.ds(r0 + (_RPT // _CH) * _CH, rem)])
        plsc.subcore_barrier()

        def step(i, carry):
            off = c * per_c + s * per_t + i * _CH
            pltpu.sync_copy(dst_hbm.at[pl.ds(off, _CH)], dstc_v)
            pltpu.sync_copy(eout_hbm.at[pl.ds(off, _CH)], rows_v)
            pltpu.sync_copy(eoutc_hbm.at[pl.ds(off, _CH)], rowsc_v)
            pltpu.sync_copy(rows_v, acc_sh.at[dstc_v], add=True)
            pltpu.sync_copy(rowsc_v, accc_sh.at[dstc_v], add=True)
            return carry

        lax.fori_loop(0, n_ch, step, 0)
        plsc.subcore_barrier()

        pltpu.sync_copy(acc_sh.at[pl.ds(r0, _RPT)],
                        agg_hbm.at[c, pl.ds(r0, _RPT)])
        pltpu.sync_copy(accc_sh.at[pl.ds(r0, _RPT)],
                        aggc_hbm.at[c, pl.ds(r0, _RPT)])

    f = pl.kernel(
        body,
        out_type=(
            jax.ShapeDtypeStruct((_NC, _ACC_R, _TW_E), BF16),
            jax.ShapeDtypeStruct((_NC, _ACC_R, _CW), F32),
        ),
        mesh=mesh,
        scratch_types=[
            pltpu.VMEM((_CH,), jnp.int32),
            pltpu.VMEM((_CH, _TW_E), BF16),
            pltpu.VMEM((_CH, _CW), F32),
            pltpu.VMEM_SHARED((_ACC_R, _TW_E), BF16),
            pltpu.VMEM_SHARED((_ACC_R, _CW), F32),
            pltpu.SemaphoreType.DMA,
        ],
        compiler_params=pltpu.CompilerParams(use_tc_tiling_on_sc=False),
    )
    return f(eout, eoutc, dst)


_BE = 2560


def _edge_mlp(gs, gd, w1c, w2, b2, w5, b5):
    def body(gs_ref, gd_ref, w1c_ref, w2_ref,
             b2_ref, w5_ref, b5_ref, out_ref, outc_ref):
        gsv = gs_ref[...]
        gdv = gd_ref[...]
        pa = gsv[:, :_HID].astype(F32)
        pb = gdv[:, :_HID].astype(F32)
        ps = gsv[:, _HID:_HID + 3].astype(F32)
        pd = gdv[:, _HID:_HID + 3].astype(F32)
        diff = ps - pd
        nrm = jnp.sqrt(jnp.sum(diff * diff, axis=-1, keepdims=True))
        dist = nrm + 1e-8
        dist2 = dist * dist
        pre = pa + pb + dist2 * w1c_ref[...]
        m = (pre * jax.nn.sigmoid(pre)).astype(BF16)
        pre2 = jnp.dot(m, w2_ref[...], preferred_element_type=F32) + b2_ref[...]
        mij = pre2 * jax.nn.sigmoid(pre2)
        wgt = jax.nn.sigmoid(
            jnp.dot(mij.astype(BF16), w5_ref[...],
                    preferred_element_type=F32) + b5_ref[...])
        out_ref[...] = mij.astype(BF16)
        outc_ref[:, :3] = wgt * (diff / dist) * 0.1
        outc_ref[:, 3:] = jnp.zeros((_BE, _CW - 3), F32)

    def wspec(r, c):
        return pl.BlockSpec((r, c), lambda i: (0, 0))

    return pl.pallas_call(
        body,
        grid=(_E // _BE,),
        in_specs=[
            pl.BlockSpec((_BE, _TW_G), lambda i: (i, 0)),
            pl.BlockSpec((_BE, _TW_G), lambda i: (i, 0)),
            wspec(1, _HID), wspec(_HID, _HID), wspec(1, _HID),
            wspec(_HID, 1), wspec(1, 1),
        ],
        out_specs=(
            pl.BlockSpec((_BE, _TW_E), lambda i: (i, 0)),
            pl.BlockSpec((_BE, _CW), lambda i: (i, 0)),
        ),
        out_shape=(
            jax.ShapeDtypeStruct((_E, _TW_E), BF16),
            jax.ShapeDtypeStruct((_E, _CW), F32),
        ),
    )(gs, gd, w1c, w2, b2, w5, b5)


_BN = 2000


def _node_mlp(h2, pos2, agg0, agg1, aggc0, aggc1, w3a, w3b, b3, w4, b4):
    def body(h_ref, pos_ref, agg0_ref, agg1_ref, aggc0_ref, aggc1_ref,
             w3a_ref, w3b_ref, b3_ref, w4_ref, b4_ref, hn_ref, xn_ref):
        hh = h_ref[...]
        magg = agg0_ref[...].astype(F32) + agg1_ref[...].astype(F32)
        upd = aggc0_ref[:, :3] + aggc1_ref[:, :3]
        pre = (jnp.dot(hh, w3a_ref[...], preferred_element_type=F32)
               + jnp.dot(magg, w3b_ref[...], preferred_element_type=F32)
               + b3_ref[...])
        nh = pre * jax.nn.sigmoid(pre)
        hn_ref[...] = hh + jnp.dot(nh, w4_ref[...],
                                   preferred_element_type=F32) + b4_ref[...]
        xn_ref[...] = pos_ref[...] + upd

    def wspec(r, c):
        return pl.BlockSpec((r, c), lambda i: (0, 0))

    return pl.pallas_call(
        body,
        grid=(_N // _BN,),
        in_specs=[
            pl.BlockSpec((_BN, _FEAT), lambda i: (i, 0)),
            pl.BlockSpec((_BN, 3), lambda i: (i, 0)),
            pl.BlockSpec((_BN, _TW_E), lambda i: (i, 0)),
            pl.BlockSpec((_BN, _TW_E), lambda i: (i, 0)),
            pl.BlockSpec((_BN, _CW), lambda i: (i, 0)),
            pl.BlockSpec((_BN, _CW), lambda i: (i, 0)),
            wspec(_FEAT, _HID), wspec(_HID, _HID), wspec(1, _HID),
            wspec(_HID, _FEAT), wspec(1, _FEAT),
        ],
        out_specs=(
            pl.BlockSpec((_BN, _FEAT), lambda i: (i, 0)),
            pl.BlockSpec((_BN, 3), lambda i: (i, 0)),
        ),
        out_shape=(
            jax.ShapeDtypeStruct((_N, _FEAT), F32),
            jax.ShapeDtypeStruct((_N, 3), F32),
        ),
    )(h2, pos2, agg0, agg1, aggc0, aggc1, w3a, w3b, b3, w4, b4)


def kernel(h, pos, edge_index, W1, b1, W2, b2, W3, b3, W4, b4, W5, b5):
    h2 = h[0]
    pos2 = pos[0]
    src = edge_index[0]
    dst = edge_index[1]

    table_a, table_b = _node_pre(
        h2, pos2, W1[:_FEAT], W1[_FEAT:2 * _FEAT], b1[None, :])
    gs, gd = _sc_gather(table_a, table_b, src, dst)

    eout, eoutc = _edge_mlp(
        gs, gd,
        W1[2 * _FEAT:2 * _FEAT + 1], W2.astype(BF16),
        b2[None, :], W5.astype(BF16), b5[None, :])

    aggp, aggc = _sc_scatter(eout, eoutc, dst)

    h_new, x_new = _node_mlp(
        h2, pos2, aggp[0, :_N], aggp[1, :_N], aggc[0, :_N], aggc[1, :_N],
        W3[:_FEAT], W3[_FEAT:], b3[None, :], W4, b4[None, :])
    return h_new[None], x_new[None]


# speedup vs baseline: 1.3546x; 1.3546x over previous
import jax
import jax.numpy as jnp
from jax import lax
from jax.experimental import pallas as pl
from jax.experimental.pallas import tpu as pltpu
from jax.experimental.pallas import tpu_sc as plsc

F32 = jnp.float32
BF16 = jnp.bfloat16

_N = 10000
_E = 320000
_FEAT = 128
_HID = 256

_TW_G = 160
_TW_E = 256
_CW = 16

_NC, _NS = 2, 16
_NW = _NC * _NS

_EC = 64000
_NCHUNK = _E // _EC

_GC = 400
_GSUB = 80
_CH = 80
_ACC_R = 10016
_RPT = _ACC_R // _NS


def _sc_gather(table, src, dst):
    per_w = _EC // _NW
    n_ch = per_w // _GC
    mesh = plsc.VectorSubcoreMesh(core_axis_name="c", subcore_axis_name="s")

    def body(table_hbm, src_hbm, dst_hbm, gs_hbm, gd_hbm,
             idxs_v, idxd_v, rs_v, rd_v, sem):
        c = lax.axis_index("c")
        s = lax.axis_index("s")
        wid = s * _NC + c
        base = wid * per_w

        def step(i, carry):
            off = base + i * _GC
            pltpu.sync_copy(src_hbm.at[pl.ds(off, _GC)], idxs_v)
            pltpu.sync_copy(dst_hbm.at[pl.ds(off, _GC)], idxd_v)
            descs = []
            for j in range(_GC // _GSUB):
                r = pl.ds(j * _GSUB, _GSUB)
                descs.append(pltpu.async_copy(
                    table_hbm.at[idxs_v.at[r]], rs_v.at[r], sem))
                descs.append(pltpu.async_copy(
                    table_hbm.at[idxd_v.at[r]], rd_v.at[r], sem))
            for d in descs:
                d.wait()
            pltpu.sync_copy(rs_v, gs_hbm.at[pl.ds(off, _GC)])
            pltpu.sync_copy(rd_v, gd_hbm.at[pl.ds(off, _GC)])
            return carry

        lax.fori_loop(0, n_ch, step, 0)

    f = pl.kernel(
        body,
        out_type=(
            jax.ShapeDtypeStruct((_EC, _TW_G), BF16),
            jax.ShapeDtypeStruct((_EC, _TW_G), BF16),
        ),
        mesh=mesh,
        scratch_types=[
            pltpu.VMEM((_GC,), jnp.int32),
            pltpu.VMEM((_GC,), jnp.int32),
            pltpu.VMEM((_GC, _TW_G), BF16),
            pltpu.VMEM((_GC, _TW_G), BF16),
            pltpu.SemaphoreType.DMA,
        ],
        compiler_params=pltpu.CompilerParams(use_tc_tiling_on_sc=False),
    )
    return f(table, src, dst)


def _sc_scatter(eouts, eoutcs, dst, e_off):
    k = len(eouts)
    per_c = _EC // _NC
    per_t = per_c // _NS
    n_ch = per_t // _CH
    mesh = plsc.VectorSubcoreMesh(core_axis_name="c", subcore_axis_name="s")

    def body(*refs):
        eo = refs[:k]
        ec = refs[k:2 * k]
        dst_hbm = refs[2 * k]
        agg_hbm = refs[2 * k + 1]
        aggc_hbm = refs[2 * k + 2]
        dstc_v, rows_v, rowsc_v, acc_sh, accc_sh, sem = refs[2 * k + 3:]
        c = lax.axis_index("c")
        s = lax.axis_index("s")

        zb16 = jnp.zeros((16,), BF16)
        zf16 = jnp.zeros((16,), F32)

        def zrow(i, carry):
            for j in range(_TW_E // 16):
                rows_v[i, pl.ds(j * 16, 16)] = zb16
            rowsc_v[i, pl.ds(0, 16)] = zf16
            return carry

        lax.fori_loop(0, _CH, zrow, 0)
        r0 = s * _RPT
        for q in range(_RPT // _CH):
            pltpu.sync_copy(rows_v, acc_sh.at[pl.ds(r0 + q * _CH, _CH)])
            pltpu.sync_copy(rowsc_v, accc_sh.at[pl.ds(r0 + q * _CH, _CH)])
        rem = _RPT - (_RPT // _CH) * _CH
        pltpu.sync_copy(rows_v.at[pl.ds(0, rem)],
                        acc_sh.at[pl.ds(r0 + (_RPT // _CH) * _CH, rem)])
        pltpu.sync_copy(rowsc_v.at[pl.ds(0, rem)],
                        accc_sh.at[pl.ds(r0 + (_RPT // _CH) * _CH, rem)])
        plsc.subcore_barrier()

        for j in range(k):
            eo_j = eo[j]
            ec_j = ec[j]

            def step(i, carry):
                loc = c * per_c + s * per_t + i * _CH
                pltpu.sync_copy(
                    dst_hbm.at[pl.ds(e_off + j * _EC + loc, _CH)], dstc_v)
                pltpu.sync_copy(eo_j.at[pl.ds(loc, _CH)], rows_v)
                pltpu.sync_copy(ec_j.at[pl.ds(loc, _CH)], rowsc_v)
                pltpu.sync_copy(rows_v, acc_sh.at[dstc_v], add=True)
                pltpu.sync_copy(rowsc_v, accc_sh.at[dstc_v], add=True)
                return carry

            lax.fori_loop(0, n_ch, step, 0)
        plsc.subcore_barrier()

        pltpu.sync_copy(acc_sh.at[pl.ds(r0, _RPT)],
                        agg_hbm.at[c, pl.ds(r0, _RPT)])
        pltpu.sync_copy(accc_sh.at[pl.ds(r0, _RPT)],
                        aggc_hbm.at[c, pl.ds(r0, _RPT)])

    f = pl.kernel(
        body,
        out_type=(
            jax.ShapeDtypeStruct((_NC, _ACC_R, _TW_E), BF16),
            jax.ShapeDtypeStruct((_NC, _ACC_R, _CW), F32),
        ),
        mesh=mesh,
        scratch_types=[
            pltpu.VMEM((_CH,), jnp.int32),
            pltpu.VMEM((_CH, _TW_E), BF16),
            pltpu.VMEM((_CH, _CW), F32),
            pltpu.VMEM_SHARED((_ACC_R, _TW_E), BF16),
            pltpu.VMEM_SHARED((_ACC_R, _CW), F32),
            pltpu.SemaphoreType.DMA,
        ],
        compiler_params=pltpu.CompilerParams(use_tc_tiling_on_sc=False),
    )
    return f(*eouts, *eoutcs, dst)


_BE = 1280


def _edge_mlp(gs, gd, w1a, w1b, w1c, b1, w2, b2, w5, b5):
    def body(gs_ref, gd_ref, w1a_ref, w1b_ref, w1c_ref, b1_ref, w2_ref,
             b2_ref, w5_ref, b5_ref, out_ref, outc_ref):
        gsv = gs_ref[...]
        gdv = gd_ref[...]
        hs = gsv[:, :_FEAT]
        hd = gdv[:, :_FEAT]
        ps = gsv[:, _FEAT:_FEAT + 3].astype(F32)
        pd = gdv[:, _FEAT:_FEAT + 3].astype(F32)
        diff = ps - pd
        nrm = jnp.sqrt(jnp.sum(diff * diff, axis=-1, keepdims=True))
        dist = nrm + 1e-8
        dist2 = dist * dist
        pre = (jnp.dot(hs, w1a_ref[...], preferred_element_type=F32)
               + jnp.dot(hd, w1b_ref[...], preferred_element_type=F32)
               + dist2 * w1c_ref[...] + b1_ref[...])
        m = (pre * jax.nn.sigmoid(pre)).astype(BF16)
        pre2 = jnp.dot(m, w2_ref[...], preferred_element_type=F32) + b2_ref[...]
        mij = pre2 * jax.nn.sigmoid(pre2)
        wgt = jax.nn.sigmoid(
            jnp.dot(mij.astype(BF16), w5_ref[...],
                    preferred_element_type=F32) + b5_ref[...])
        out_ref[...] = mij.astype(BF16)
        outc_ref[:, :3] = wgt * (diff / dist) * 0.1
        outc_ref[:, 3:] = jnp.zeros((_BE, _CW - 3), F32)

    def wspec(r, c):
        return pl.BlockSpec((r, c), lambda i: (0, 0))

    return pl.pallas_call(
        body,
        grid=(_EC // _BE,),
        in_specs=[
            pl.BlockSpec((_BE, _TW_G), lambda i: (i, 0)),
            pl.BlockSpec((_BE, _TW_G), lambda i: (i, 0)),
            wspec(_FEAT, _HID), wspec(_FEAT, _HID), wspec(1, _HID),
            wspec(1, _HID), wspec(_HID, _HID), wspec(1, _HID),
            wspec(_HID, 1), wspec(1, 1),
        ],
        out_specs=(
            pl.BlockSpec((_BE, _TW_E), lambda i: (i, 0)),
            pl.BlockSpec((_BE, _CW), lambda i: (i, 0)),
        ),
        out_shape=(
            jax.ShapeDtypeStruct((_EC, _TW_E), BF16),
            jax.ShapeDtypeStruct((_EC, _CW), F32),
        ),
    )(gs, gd, w1a, w1b, w1c, b1, w2, b2, w5, b5)


_BN = 2000


def _node_mlp(h2, pos2, aggs, aggcs, w3a, w3b, b3, w4, b4):
    n_p = len(aggs)

    def body(*refs):
        h_ref = refs[0]
        pos_ref = refs[1]
        agg_refs = refs[2:2 + n_p]
        aggc_refs = refs[2 + n_p:2 + 2 * n_p]
        w3a_ref, w3b_ref, b3_ref, w4_ref, b4_ref = refs[2 + 2 * n_p:
                                                        7 + 2 * n_p]
        hn_ref, xn_ref = refs[7 + 2 * n_p:]
        hh = h_ref[...]
        magg = agg_refs[0][...].astype(F32)
        for r in agg_refs[1:]:
            magg = magg + r[...].astype(F32)
        upd = aggc_refs[0][:, :3]
        for r in aggc_refs[1:]:
            upd = upd + r[:, :3]
        pre = (jnp.dot(hh, w3a_ref[...], preferred_element_type=F32)
               + jnp.dot(magg, w3b_ref[...], preferred_element_type=F32)
               + b3_ref[...])
        nh = pre * jax.nn.sigmoid(pre)
        hn_ref[...] = hh + jnp.dot(nh, w4_ref[...],
                                   preferred_element_type=F32) + b4_ref[...]
        xn_ref[...] = pos_ref[...] + upd

    def wspec(r, c):
        return pl.BlockSpec((r, c), lambda i: (0, 0))

    return pl.pallas_call(
        body,
        grid=(_N // _BN,),
        in_specs=[
            pl.BlockSpec((_BN, _FEAT), lambda i: (i, 0)),
            pl.BlockSpec((_BN, 3), lambda i: (i, 0)),
        ] + [pl.BlockSpec((_BN, _TW_E), lambda i: (i, 0))] * n_p
          + [pl.BlockSpec((_BN, _CW), lambda i: (i, 0))] * n_p
          + [wspec(_FEAT, _HID), wspec(_HID, _HID), wspec(1, _HID),
             wspec(_HID, _FEAT), wspec(1, _FEAT)],
        out_specs=(
            pl.BlockSpec((_BN, _FEAT), lambda i: (i, 0)),
            pl.BlockSpec((_BN, 3), lambda i: (i, 0)),
        ),
        out_shape=(
            jax.ShapeDtypeStruct((_N, _FEAT), F32),
            jax.ShapeDtypeStruct((_N, 3), F32),
        ),
    )(h2, pos2, *aggs, *aggcs, w3a, w3b, b3, w4, b4)


def kernel(h, pos, edge_index, W1, b1, W2, b2, W3, b3, W4, b4, W5, b5):
    h2 = h[0]
    pos2 = pos[0]
    src = edge_index[0]
    dst = edge_index[1]

    table = jnp.concatenate(
        [h2.astype(BF16), pos2.astype(BF16),
         jnp.zeros((_N, _TW_G - _FEAT - 3), BF16)], axis=1)

    w1a = W1[:_FEAT].astype(BF16)
    w1b = W1[_FEAT:2 * _FEAT].astype(BF16)
    w1c = W1[2 * _FEAT:2 * _FEAT + 1]
    w2 = W2.astype(BF16)
    w5 = W5.astype(BF16)

    eouts, eoutcs = [], []
    for kk in range(_NCHUNK):
        sl = slice(kk * _EC, (kk + 1) * _EC)
        gs, gd = _sc_gather(table, src[sl], dst[sl])
        eo, ec = _edge_mlp(gs, gd, w1a, w1b, w1c, b1[None, :], w2,
                           b2[None, :], w5, b5[None, :])
        eouts.append(eo)
        eoutcs.append(ec)

    aggp0, aggc0 = _sc_scatter(eouts[:2], eoutcs[:2], dst, 0)
    aggp1, aggc1 = _sc_scatter(eouts[2:], eoutcs[2:], dst, 2 * _EC)

    aggs = [aggp0[0, :_N], aggp0[1, :_N], aggp1[0, :_N], aggp1[1, :_N]]
    aggcs = [aggc0[0, :_N], aggc0[1, :_N], aggc1[0, :_N], aggc1[1, :_N]]

    h_new, x_new = _node_mlp(
        h2, pos2, aggs, aggcs,
        W3[:_FEAT], W3[_FEAT:], b3[None, :], W4, b4[None, :])
    return h_new[None], x_new[None]


# speedup vs baseline: 1.6469x; 1.2158x over previous
import jax
import jax.numpy as jnp
from jax import lax
from jax.experimental import pallas as pl
from jax.experimental.pallas import tpu as pltpu
from jax.experimental.pallas import tpu_sc as plsc

F32 = jnp.float32
BF16 = jnp.bfloat16

_N = 10000
_E = 320000
_FEAT = 128
_HID = 256

_TW_G = 128
_TW_E = 256
_CW = 16

_NC, _NS = 2, 16
_NW = _NC * _NS

_EC = 64000
_NCHUNK = _E // _EC

_GC = 200
_GSUB = 40
_CH = 80
_ACC_R = 10016
_RPT = _ACC_R // _NS


def _sc_gather(table, src, dst):
    per_w = _EC // _NW
    n_ch = per_w // _GC
    mesh = plsc.VectorSubcoreMesh(core_axis_name="c", subcore_axis_name="s")

    def body(table_hbm, src_hbm, dst_hbm, gs_hbm, gd_hbm,
             idxs_v, idxd_v, rs_v, rd_v, sem):
        c = lax.axis_index("c")
        s = lax.axis_index("s")
        wid = s * _NC + c
        base = wid * per_w

        def step(i, carry):
            off = base + i * _GC
            pltpu.sync_copy(src_hbm.at[pl.ds(off, _GC)], idxs_v)
            pltpu.sync_copy(dst_hbm.at[pl.ds(off, _GC)], idxd_v)
            descs = []
            for j in range(_GC // _GSUB):
                r = pl.ds(j * _GSUB, _GSUB)
                descs.append(pltpu.async_copy(
                    table_hbm.at[idxs_v.at[r]], rs_v.at[r], sem))
                descs.append(pltpu.async_copy(
                    table_hbm.at[idxd_v.at[r]], rd_v.at[r], sem))
            for d in descs:
                d.wait()
            pltpu.sync_copy(rs_v, gs_hbm.at[pl.ds(off, _GC)])
            pltpu.sync_copy(rd_v, gd_hbm.at[pl.ds(off, _GC)])
            return carry

        lax.fori_loop(0, n_ch, step, 0)

    f = pl.kernel(
        body,
        out_type=(
            jax.ShapeDtypeStruct((_EC, _TW_G), F32),
            jax.ShapeDtypeStruct((_EC, _TW_G), F32),
        ),
        mesh=mesh,
        scratch_types=[
            pltpu.VMEM((_GC,), jnp.int32),
            pltpu.VMEM((_GC,), jnp.int32),
            pltpu.VMEM((_GC, _TW_G), F32),
            pltpu.VMEM((_GC, _TW_G), F32),
            pltpu.SemaphoreType.DMA,
        ],
    )
    return f(table, src, dst)


def _sc_gather_pos(pos16, src, dst):
    per_w = _E // _NW
    n_ch = per_w // _GC
    mesh = plsc.VectorSubcoreMesh(core_axis_name="c", subcore_axis_name="s")

    def body(pos_hbm, src_hbm, dst_hbm, ps_hbm, pd_hbm,
             idxs_v, idxd_v, rs_v, rd_v, sem):
        c = lax.axis_index("c")
        s = lax.axis_index("s")
        wid = s * _NC + c
        base = wid * per_w

        def step(i, carry):
            off = base + i * _GC
            pltpu.sync_copy(src_hbm.at[pl.ds(off, _GC)], idxs_v)
            pltpu.sync_copy(dst_hbm.at[pl.ds(off, _GC)], idxd_v)
            descs = []
            for j in range(_GC // _GSUB):
                r = pl.ds(j * _GSUB, _GSUB)
                descs.append(pltpu.async_copy(
                    pos_hbm.at[idxs_v.at[r]], rs_v.at[r], sem))
                descs.append(pltpu.async_copy(
                    pos_hbm.at[idxd_v.at[r]], rd_v.at[r], sem))
            for d in descs:
                d.wait()
            pltpu.sync_copy(rs_v, ps_hbm.at[pl.ds(off, _GC)])
            pltpu.sync_copy(rd_v, pd_hbm.at[pl.ds(off, _GC)])
            return carry

        lax.fori_loop(0, n_ch, step, 0)

    f = pl.kernel(
        body,
        out_type=(
            jax.ShapeDtypeStruct((_E, _CW), F32),
            jax.ShapeDtypeStruct((_E, _CW), F32),
        ),
        mesh=mesh,
        scratch_types=[
            pltpu.VMEM((_GC,), jnp.int32),
            pltpu.VMEM((_GC,), jnp.int32),
            pltpu.VMEM((_GC, _CW), F32),
            pltpu.VMEM((_GC, _CW), F32),
            pltpu.SemaphoreType.DMA,
        ],
        compiler_params=pltpu.CompilerParams(use_tc_tiling_on_sc=False),
    )
    return f(pos16, src, dst)


def _sc_scatter_m(eouts, dst):
    k = len(eouts)
    per_c = _EC // _NC
    per_t = per_c // _NS
    n_ch = per_t // _CH
    mesh = plsc.VectorSubcoreMesh(core_axis_name="c", subcore_axis_name="s")

    def body(*refs):
        eo = refs[:k]
        dst_hbm = refs[k]
        agg_hbm = refs[k + 1]
        dstc_v, rows_v, acc_sh, sem = refs[k + 2:]
        c = lax.axis_index("c")
        s = lax.axis_index("s")

        zb16 = jnp.zeros((16,), BF16)

        def zrow(i, carry):
            for j in range(_TW_E // 16):
                rows_v[i, pl.ds(j * 16, 16)] = zb16
            return carry

        lax.fori_loop(0, _CH, zrow, 0)
        r0 = s * _RPT
        for q in range(_RPT // _CH):
            pltpu.sync_copy(rows_v, acc_sh.at[pl.ds(r0 + q * _CH, _CH)])
        rem = _RPT - (_RPT // _CH) * _CH
        pltpu.sync_copy(rows_v.at[pl.ds(0, rem)],
                        acc_sh.at[pl.ds(r0 + (_RPT // _CH) * _CH, rem)])
        plsc.subcore_barrier()

        for j in range(k):
            eo_j = eo[j]

            def step(i, carry):
                loc = c * per_c + s * per_t + i * _CH
                pltpu.sync_copy(dst_hbm.at[pl.ds(j * _EC + loc, _CH)], dstc_v)
                pltpu.sync_copy(eo_j.at[pl.ds(loc, _CH)], rows_v)
                pltpu.sync_copy(rows_v, acc_sh.at[dstc_v], add=True)
                return carry

            lax.fori_loop(0, n_ch, step, 0)
        plsc.subcore_barrier()

        pltpu.sync_copy(acc_sh.at[pl.ds(r0, _RPT)],
                        agg_hbm.at[c, pl.ds(r0, _RPT)])

    f = pl.kernel(
        body,
        out_type=jax.ShapeDtypeStruct((_NC, _ACC_R, _TW_E), BF16),
        mesh=mesh,
        scratch_types=[
            pltpu.VMEM((_CH,), jnp.int32),
            pltpu.VMEM((_CH, _TW_E), BF16),
            pltpu.VMEM_SHARED((_ACC_R, _TW_E), BF16),
            pltpu.SemaphoreType.DMA,
        ],
        compiler_params=pltpu.CompilerParams(use_tc_tiling_on_sc=False),
    )
    return f(*eouts, dst)


def _sc_scatter_c(eoutcs, dst):
    k = len(eoutcs)
    per_c = _EC // _NC
    per_t = per_c // _NS
    n_ch = per_t // _CH
    mesh = plsc.VectorSubcoreMesh(core_axis_name="c", subcore_axis_name="s")

    def body(*refs):
        ec = refs[:k]
        dst_hbm = refs[k]
        aggc_hbm = refs[k + 1]
        dstc_v, rowsc_v, accc_sh, sem = refs[k + 2:]
        c = lax.axis_index("c")
        s = lax.axis_index("s")

        zf16 = jnp.zeros((16,), F32)

        def zrow(i, carry):
            rowsc_v[i, pl.ds(0, 16)] = zf16
            return carry

        lax.fori_loop(0, _CH, zrow, 0)
        r0 = s * _RPT
        for q in range(_RPT // _CH):
            pltpu.sync_copy(rowsc_v, accc_sh.at[pl.ds(r0 + q * _CH, _CH)])
        rem = _RPT - (_RPT // _CH) * _CH
        pltpu.sync_copy(rowsc_v.at[pl.ds(0, rem)],
                        accc_sh.at[pl.ds(r0 + (_RPT // _CH) * _CH, rem)])
        plsc.subcore_barrier()

        for j in range(k):
            ec_j = ec[j]

            def step(i, carry):
                loc = c * per_c + s * per_t + i * _CH
                pltpu.sync_copy(dst_hbm.at[pl.ds(j * _EC + loc, _CH)], dstc_v)
                pltpu.sync_copy(ec_j.at[pl.ds(loc, _CH)], rowsc_v)
                pltpu.sync_copy(rowsc_v, accc_sh.at[dstc_v], add=True)
                return carry

            lax.fori_loop(0, n_ch, step, 0)
        plsc.subcore_barrier()

        pltpu.sync_copy(accc_sh.at[pl.ds(r0, _RPT)],
                        aggc_hbm.at[c, pl.ds(r0, _RPT)])

    f = pl.kernel(
        body,
        out_type=jax.ShapeDtypeStruct((_NC, _ACC_R, _CW), F32),
        mesh=mesh,
        scratch_types=[
            pltpu.VMEM((_CH,), jnp.int32),
            pltpu.VMEM((_CH, _CW), F32),
            pltpu.VMEM_SHARED((_ACC_R, _CW), F32),
            pltpu.SemaphoreType.DMA,
        ],
        compiler_params=pltpu.CompilerParams(use_tc_tiling_on_sc=False),
    )
    return f(*eoutcs, dst)


_BE = 1280


def _edge_mlp(gs, gd, ps16, pd16, w1a, w1b, w1c, b1, w2, b2, w5, b5):
    def body(gs_ref, gd_ref, ps_ref, pd_ref, w1a_ref, w1b_ref, w1c_ref,
             b1_ref, w2_ref, b2_ref, w5_ref, b5_ref, out_ref, outc_ref):
        hs = gs_ref[...].astype(BF16)
        hd = gd_ref[...].astype(BF16)
        ps = ps_ref[:, :3]
        pd = pd_ref[:, :3]
        diff = ps - pd
        nrm = jnp.sqrt(jnp.sum(diff * diff, axis=-1, keepdims=True))
        dist = nrm + 1e-8
        dist2 = dist * dist
        pre = (jnp.dot(hs, w1a_ref[...], preferred_element_type=F32)
               + jnp.dot(hd, w1b_ref[...], preferred_element_type=F32)
               + dist2 * w1c_ref[...] + b1_ref[...])
        m = (pre * jax.nn.sigmoid(pre)).astype(BF16)
        pre2 = jnp.dot(m, w2_ref[...], preferred_element_type=F32) + b2_ref[...]
        mij = pre2 * jax.nn.sigmoid(pre2)
        wgt = jax.nn.sigmoid(
            jnp.dot(mij.astype(BF16), w5_ref[...],
                    preferred_element_type=F32) + b5_ref[...])
        out_ref[...] = mij.astype(BF16)
        outc_ref[:, :3] = wgt * (diff / dist) * 0.1
        outc_ref[:, 3:] = jnp.zeros((_BE, _CW - 3), F32)

    def wspec(r, c):
        return pl.BlockSpec((r, c), lambda i: (0, 0))

    return pl.pallas_call(
        body,
        grid=(_EC // _BE,),
        in_specs=[
            pl.BlockSpec((_BE, _TW_G), lambda i: (i, 0)),
            pl.BlockSpec((_BE, _TW_G), lambda i: (i, 0)),
            pl.BlockSpec((_BE, _CW), lambda i: (i, 0)),
            pl.BlockSpec((_BE, _CW), lambda i: (i, 0)),
            wspec(_FEAT, _HID), wspec(_FEAT, _HID), wspec(1, _HID),
            wspec(1, _HID), wspec(_HID, _HID), wspec(1, _HID),
            wspec(_HID, 1), wspec(1, 1),
        ],
        out_specs=(
            pl.BlockSpec((_BE, _TW_E), lambda i: (i, 0)),
            pl.BlockSpec((_BE, _CW), lambda i: (i, 0)),
        ),
        out_shape=(
            jax.ShapeDtypeStruct((_EC, _TW_E), BF16),
            jax.ShapeDtypeStruct((_EC, _CW), F32),
        ),
    )(gs, gd, ps16, pd16, w1a, w1b, w1c, b1, w2, b2, w5, b5)


_BN = 2000


def _node_mlp(h2, pos2, aggs, aggcs, w3a, w3b, b3, w4, b4):
    n_p = len(aggs)

    def body(*refs):
        h_ref = refs[0]
        pos_ref = refs[1]
        agg_refs = refs[2:2 + n_p]
        aggc_refs = refs[2 + n_p:2 + 2 * n_p]
        w3a_ref, w3b_ref, b3_ref, w4_ref, b4_ref = refs[2 + 2 * n_p:
                                                        7 + 2 * n_p]
        hn_ref, xn_ref = refs[7 + 2 * n_p:]
        hh = h_ref[...]
        magg = agg_refs[0][...].astype(F32)
        for r in agg_refs[1:]:
            magg = magg + r[...].astype(F32)
        upd = aggc_refs[0][:, :3]
        for r in aggc_refs[1:]:
            upd = upd + r[:, :3]
        pre = (jnp.dot(hh, w3a_ref[...], preferred_element_type=F32)
               + jnp.dot(magg, w3b_ref[...], preferred_element_type=F32)
               + b3_ref[...])
        nh = pre * jax.nn.sigmoid(pre)
        hn_ref[...] = hh + jnp.dot(nh, w4_ref[...],
                                   preferred_element_type=F32) + b4_ref[...]
        xn_ref[...] = pos_ref[...] + upd

    def wspec(r, c):
        return pl.BlockSpec((r, c), lambda i: (0, 0))

    return pl.pallas_call(
        body,
        grid=(_N // _BN,),
        in_specs=[
            pl.BlockSpec((_BN, _FEAT), lambda i: (i, 0)),
            pl.BlockSpec((_BN, 3), lambda i: (i, 0)),
        ] + [pl.BlockSpec((_BN, _TW_E), lambda i: (i, 0))] * n_p
          + [pl.BlockSpec((_BN, _CW), lambda i: (i, 0))] * n_p
          + [wspec(_FEAT, _HID), wspec(_HID, _HID), wspec(1, _HID),
             wspec(_HID, _FEAT), wspec(1, _FEAT)],
        out_specs=(
            pl.BlockSpec((_BN, _FEAT), lambda i: (i, 0)),
            pl.BlockSpec((_BN, 3), lambda i: (i, 0)),
        ),
        out_shape=(
            jax.ShapeDtypeStruct((_N, _FEAT), F32),
            jax.ShapeDtypeStruct((_N, 3), F32),
        ),
    )(h2, pos2, *aggs, *aggcs, w3a, w3b, b3, w4, b4)


def kernel(h, pos, edge_index, W1, b1, W2, b2, W3, b3, W4, b4, W5, b5):
    h2 = h[0]
    pos2 = pos[0]
    src = edge_index[0]
    dst = edge_index[1]

    pos16 = jnp.concatenate(
        [pos2, jnp.zeros((_N, _CW - 3), F32)], axis=1)

    w1a = W1[:_FEAT].astype(BF16)
    w1b = W1[_FEAT:2 * _FEAT].astype(BF16)
    w1c = W1[2 * _FEAT:2 * _FEAT + 1]
    w2 = W2.astype(BF16)
    w5 = W5.astype(BF16)

    ps16, pd16 = _sc_gather_pos(pos16, src, dst)

    eouts, eoutcs = [], []
    for kk in range(_NCHUNK):
        sl = slice(kk * _EC, (kk + 1) * _EC)
        gs, gd = _sc_gather(h2, src[sl], dst[sl])
        eo, ec = _edge_mlp(gs, gd, ps16[sl], pd16[sl], w1a, w1b, w1c,
                           b1[None, :], w2, b2[None, :], w5, b5[None, :])
        eouts.append(eo)
        eoutcs.append(ec)

    aggp = _sc_scatter_m(eouts, dst)
    aggc = _sc_scatter_c(eoutcs, dst)

    aggs = [aggp[0, :_N], aggp[1, :_N]]
    aggcs = [aggc[0, :_N], aggc[1, :_N]]

    h_new, x_new = _node_mlp(
        h2, pos2, aggs, aggcs,
        W3[:_FEAT], W3[_FEAT:], b3[None, :], W4, b4[None, :])
    return h_new[None], x_new[None]


# speedup vs baseline: 1.8347x; 1.1140x over previous
import jax
import jax.numpy as jnp
from jax import lax
from jax.experimental import pallas as pl
from jax.experimental.pallas import tpu as pltpu
from jax.experimental.pallas import tpu_sc as plsc

F32 = jnp.float32
BF16 = jnp.bfloat16

_N = 10000
_E = 320000
_FEAT = 128
_HID = 256

_TW_G = 128
_TW_E = 256
_CW = 16

_NC, _NS = 2, 16
_NW = _NC * _NS

_EC = 64000
_NCHUNK = _E // _EC

_GC = 80
_GSUB = 40
_GCP = 2000
_GSUBP = 80
_CH = 80
_ACC_R = 10016
_RPT = _ACC_R // _NS


def _sc_gather(table, src, dst):
    per_w = _EC // _NW
    n_ch = per_w // _GC
    mesh = plsc.VectorSubcoreMesh(core_axis_name="c", subcore_axis_name="s")

    def body(table_hbm, src_hbm, dst_hbm, gs_hbm, gd_hbm,
             idxs_v, idxd_v, rs_v, rd_v, table_sh, sem):
        c = lax.axis_index("c")
        s = lax.axis_index("s")
        wid = s * _NC + c
        base = wid * per_w

        rpt = 624
        pltpu.sync_copy(table_hbm.at[pl.ds(s * rpt, rpt)],
                        table_sh.at[pl.ds(s * rpt, rpt)])
        pltpu.sync_copy(table_hbm.at[pl.ds(_NS * rpt, _N - _NS * rpt)],
                        table_sh.at[pl.ds(_NS * rpt, _N - _NS * rpt)])
        plsc.subcore_barrier()

        def step(i, carry):
            off = base + i * _GC
            pltpu.sync_copy(src_hbm.at[pl.ds(off, _GC)], idxs_v)
            pltpu.sync_copy(dst_hbm.at[pl.ds(off, _GC)], idxd_v)
            descs = []
            for j in range(_GC // _GSUB):
                r = pl.ds(j * _GSUB, _GSUB)
                descs.append(pltpu.async_copy(
                    table_sh.at[idxs_v.at[r]], rs_v.at[r], sem))
                descs.append(pltpu.async_copy(
                    table_sh.at[idxd_v.at[r]], rd_v.at[r], sem))
            for d in descs:
                d.wait()
            pltpu.sync_copy(rs_v, gs_hbm.at[pl.ds(off, _GC)])
            pltpu.sync_copy(rd_v, gd_hbm.at[pl.ds(off, _GC)])
            return carry

        lax.fori_loop(0, n_ch, step, 0)

    f = pl.kernel(
        body,
        out_type=(
            jax.ShapeDtypeStruct((_EC, _TW_G), F32),
            jax.ShapeDtypeStruct((_EC, _TW_G), F32),
        ),
        mesh=mesh,
        scratch_types=[
            pltpu.VMEM((_GC,), jnp.int32),
            pltpu.VMEM((_GC,), jnp.int32),
            pltpu.VMEM((_GC, _TW_G), F32),
            pltpu.VMEM((_GC, _TW_G), F32),
            pltpu.VMEM_SHARED((_N, _TW_G), F32),
            pltpu.SemaphoreType.DMA,
        ],
    )
    return f(table, src, dst)


def _sc_gather_pos(pos16, src, dst):
    per_w = _E // _NW
    n_ch = per_w // _GCP
    mesh = plsc.VectorSubcoreMesh(core_axis_name="c", subcore_axis_name="s")

    def body(pos_hbm, src_hbm, dst_hbm, ps_hbm, pd_hbm,
             idxs_v, idxd_v, rs_v, rd_v, sem):
        c = lax.axis_index("c")
        s = lax.axis_index("s")
        wid = s * _NC + c
        base = wid * per_w

        def step(i, carry):
            off = base + i * _GCP
            pltpu.sync_copy(src_hbm.at[pl.ds(off, _GCP)], idxs_v)
            pltpu.sync_copy(dst_hbm.at[pl.ds(off, _GCP)], idxd_v)
            descs = []
            for j in range(_GCP // _GSUBP):
                r = pl.ds(j * _GSUBP, _GSUBP)
                descs.append(pltpu.async_copy(
                    pos_hbm.at[idxs_v.at[r]], rs_v.at[r], sem))
                descs.append(pltpu.async_copy(
                    pos_hbm.at[idxd_v.at[r]], rd_v.at[r], sem))
            for d in descs:
                d.wait()
            pltpu.sync_copy(rs_v, ps_hbm.at[pl.ds(off, _GCP)])
            pltpu.sync_copy(rd_v, pd_hbm.at[pl.ds(off, _GCP)])
            return carry

        lax.fori_loop(0, n_ch, step, 0)

    f = pl.kernel(
        body,
        out_type=(
            jax.ShapeDtypeStruct((_E, _CW), F32),
            jax.ShapeDtypeStruct((_E, _CW), F32),
        ),
        mesh=mesh,
        scratch_types=[
            pltpu.VMEM((_GCP,), jnp.int32),
            pltpu.VMEM((_GCP,), jnp.int32),
            pltpu.VMEM((_GCP, _CW), F32),
            pltpu.VMEM((_GCP, _CW), F32),
            pltpu.SemaphoreType.DMA,
        ],
        compiler_params=pltpu.CompilerParams(use_tc_tiling_on_sc=False),
    )
    return f(pos16, src, dst)


def _sc_scatter(eouts, eoutcs, dst):
    k = len(eouts)
    per_c = _EC // _NC
    per_t = per_c // _NS
    n_ch = per_t // _CH
    mesh = plsc.VectorSubcoreMesh(core_axis_name="c", subcore_axis_name="s")

    def body(*refs):
        eo = refs[:k]
        ec = refs[k:2 * k]
        dst_hbm = refs[2 * k]
        agg_hbm = refs[2 * k + 1]
        aggc_hbm = refs[2 * k + 2]
        dstc_v, rows_v, rowsc_v, acc_sh, accc_sh, sem = refs[2 * k + 3:]
        c = lax.axis_index("c")
        s = lax.axis_index("s")

        zb16 = jnp.zeros((16,), BF16)
        zf16 = jnp.zeros((16,), F32)

        def zrow(i, carry):
            for j in range(_TW_E // 16):
                rows_v[i, pl.ds(j * 16, 16)] = zb16
            rowsc_v[i, pl.ds(0, 16)] = zf16
            return carry

        lax.fori_loop(0, _CH, zrow, 0)
        r0 = s * _RPT
        for q in range(_RPT // _CH):
            pltpu.sync_copy(rows_v, acc_sh.at[pl.ds(r0 + q * _CH, _CH)])
            pltpu.sync_copy(rowsc_v, accc_sh.at[pl.ds(r0 + q * _CH, _CH)])
        rem = _RPT - (_RPT // _CH) * _CH
        pltpu.sync_copy(rows_v.at[pl.ds(0, rem)],
                        acc_sh.at[pl.ds(r0 + (_RPT // _CH) * _CH, rem)])
        pltpu.sync_copy(rowsc_v.at[pl.ds(0, rem)],
                        accc_sh.at[pl.ds(r0 + (_RPT // _CH) * _CH, rem)])
        plsc.subcore_barrier()

        for j in range(k):
            eo_j = eo[j]
            ec_j = ec[j]

            def step(i, carry):
                loc = c * per_c + s * per_t + i * _CH
                pltpu.sync_copy(dst_hbm.at[pl.ds(j * _EC + loc, _CH)], dstc_v)
                pltpu.sync_copy(eo_j.at[pl.ds(loc, _CH)], rows_v)
                pltpu.sync_copy(ec_j.at[pl.ds(loc, _CH)], rowsc_v)
                pltpu.sync_copy(rows_v, acc_sh.at[dstc_v], add=True)
                pltpu.sync_copy(rowsc_v, accc_sh.at[dstc_v], add=True)
                return carry

            lax.fori_loop(0, n_ch, step, 0)
        plsc.subcore_barrier()

        pltpu.sync_copy(acc_sh.at[pl.ds(r0, _RPT)],
                        agg_hbm.at[c, pl.ds(r0, _RPT)])
        pltpu.sync_copy(accc_sh.at[pl.ds(r0, _RPT)],
                        aggc_hbm.at[c, pl.ds(r0, _RPT)])

    f = pl.kernel(
        body,
        out_type=(
            jax.ShapeDtypeStruct((_NC, _ACC_R, _TW_E), BF16),
            jax.ShapeDtypeStruct((_NC, _ACC_R, _CW), F32),
        ),
        mesh=mesh,
        scratch_types=[
            pltpu.VMEM((_CH,), jnp.int32),
            pltpu.VMEM((_CH, _TW_E), BF16),
            pltpu.VMEM((_CH, _CW), F32),
            pltpu.VMEM_SHARED((_ACC_R, _TW_E), BF16),
            pltpu.VMEM_SHARED((_ACC_R, _CW), F32),
            pltpu.SemaphoreType.DMA,
        ],
        compiler_params=pltpu.CompilerParams(use_tc_tiling_on_sc=False),
    )
    return f(*eouts, *eoutcs, dst)


_BE = 1280


def _edge_mlp(gs, gd, ps16, pd16, w1a, w1b, w1c, b1, w2, b2, w5, b5):
    def body(gs_ref, gd_ref, ps_ref, pd_ref, w1a_ref, w1b_ref, w1c_ref,
             b1_ref, w2_ref, b2_ref, w5_ref, b5_ref, out_ref, outc_ref):
        hs = gs_ref[...].astype(BF16)
        hd = gd_ref[...].astype(BF16)
        ps = ps_ref[:, :3]
        pd = pd_ref[:, :3]
        diff = ps - pd
        nrm = jnp.sqrt(jnp.sum(diff * diff, axis=-1, keepdims=True))
        dist = nrm + 1e-8
        dist2 = dist * dist
        pre = (jnp.dot(hs, w1a_ref[...], preferred_element_type=F32)
               + jnp.dot(hd, w1b_ref[...], preferred_element_type=F32)
               + dist2 * w1c_ref[...] + b1_ref[...])
        m = (pre * jax.nn.sigmoid(pre)).astype(BF16)
        pre2 = jnp.dot(m, w2_ref[...], preferred_element_type=F32) + b2_ref[...]
        mij = pre2 * jax.nn.sigmoid(pre2)
        wgt = jax.nn.sigmoid(
            jnp.dot(mij.astype(BF16), w5_ref[...],
                    preferred_element_type=F32) + b5_ref[...])
        out_ref[...] = mij.astype(BF16)
        outc_ref[:, :3] = wgt * (diff / dist) * 0.1
        outc_ref[:, 3:] = jnp.zeros((_BE, _CW - 3), F32)

    def wspec(r, c):
        return pl.BlockSpec((r, c), lambda i: (0, 0))

    return pl.pallas_call(
        body,
        grid=(_EC // _BE,),
        in_specs=[
            pl.BlockSpec((_BE, _TW_G), lambda i: (i, 0)),
            pl.BlockSpec((_BE, _TW_G), lambda i: (i, 0)),
            pl.BlockSpec((_BE, _CW), lambda i: (i, 0)),
            pl.BlockSpec((_BE, _CW), lambda i: (i, 0)),
            wspec(_FEAT, _HID), wspec(_FEAT, _HID), wspec(1, _HID),
            wspec(1, _HID), wspec(_HID, _HID), wspec(1, _HID),
            wspec(_HID, 1), wspec(1, 1),
        ],
        out_specs=(
            pl.BlockSpec((_BE, _TW_E), lambda i: (i, 0)),
            pl.BlockSpec((_BE, _CW), lambda i: (i, 0)),
        ),
        out_shape=(
            jax.ShapeDtypeStruct((_EC, _TW_E), BF16),
            jax.ShapeDtypeStruct((_EC, _CW), F32),
        ),
    )(gs, gd, ps16, pd16, w1a, w1b, w1c, b1, w2, b2, w5, b5)


_BN = 2000


def _node_mlp(h2, pos2, aggs, aggcs, w3a, w3b, b3, w4, b4):
    n_p = len(aggs)

    def body(*refs):
        h_ref = refs[0]
        pos_ref = refs[1]
        agg_refs = refs[2:2 + n_p]
        aggc_refs = refs[2 + n_p:2 + 2 * n_p]
        w3a_ref, w3b_ref, b3_ref, w4_ref, b4_ref = refs[2 + 2 * n_p:
                                                        7 + 2 * n_p]
        hn_ref, xn_ref = refs[7 + 2 * n_p:]
        hh = h_ref[...]
        magg = agg_refs[0][...].astype(F32)
        for r in agg_refs[1:]:
            magg = magg + r[...].astype(F32)
        upd = aggc_refs[0][:, :3]
        for r in aggc_refs[1:]:
            upd = upd + r[:, :3]
        pre = (jnp.dot(hh, w3a_ref[...], preferred_element_type=F32)
               + jnp.dot(magg, w3b_ref[...], preferred_element_type=F32)
               + b3_ref[...])
        nh = pre * jax.nn.sigmoid(pre)
        hn_ref[...] = hh + jnp.dot(nh, w4_ref[...],
                                   preferred_element_type=F32) + b4_ref[...]
        xn_ref[...] = pos_ref[...] + upd

    def wspec(r, c):
        return pl.BlockSpec((r, c), lambda i: (0, 0))

    return pl.pallas_call(
        body,
        grid=(_N // _BN,),
        in_specs=[
            pl.BlockSpec((_BN, _FEAT), lambda i: (i, 0)),
            pl.BlockSpec((_BN, 3), lambda i: (i, 0)),
        ] + [pl.BlockSpec((_BN, _TW_E), lambda i: (i, 0))] * n_p
          + [pl.BlockSpec((_BN, _CW), lambda i: (i, 0))] * n_p
          + [wspec(_FEAT, _HID), wspec(_HID, _HID), wspec(1, _HID),
             wspec(_HID, _FEAT), wspec(1, _FEAT)],
        out_specs=(
            pl.BlockSpec((_BN, _FEAT), lambda i: (i, 0)),
            pl.BlockSpec((_BN, 3), lambda i: (i, 0)),
        ),
        out_shape=(
            jax.ShapeDtypeStruct((_N, _FEAT), F32),
            jax.ShapeDtypeStruct((_N, 3), F32),
        ),
    )(h2, pos2, *aggs, *aggcs, w3a, w3b, b3, w4, b4)


def kernel(h, pos, edge_index, W1, b1, W2, b2, W3, b3, W4, b4, W5, b5):
    h2 = h[0]
    pos2 = pos[0]
    src = edge_index[0]
    dst = edge_index[1]

    pos16 = jnp.concatenate(
        [pos2, jnp.zeros((_N, _CW - 3), F32)], axis=1)

    w1a = W1[:_FEAT].astype(BF16)
    w1b = W1[_FEAT:2 * _FEAT].astype(BF16)
    w1c = W1[2 * _FEAT:2 * _FEAT + 1]
    w2 = W2.astype(BF16)
    w5 = W5.astype(BF16)

    ps16, pd16 = _sc_gather_pos(pos16, src, dst)

    eouts, eoutcs = [], []
    for kk in range(_NCHUNK):
        sl = slice(kk * _EC, (kk + 1) * _EC)
        gs, gd = _sc_gather(h2, src[sl], dst[sl])
        eo, ec = _edge_mlp(gs, gd, ps16[sl], pd16[sl], w1a, w1b, w1c,
                           b1[None, :], w2, b2[None, :], w5, b5[None, :])
        eouts.append(eo)
        eoutcs.append(ec)

    aggp, aggc = _sc_scatter(eouts, eoutcs, dst)

    aggs = [aggp[0, :_N], aggp[1, :_N]]
    aggcs = [aggc[0, :_N], aggc[1, :_N]]

    h_new, x_new = _node_mlp(
        h2, pos2, aggs, aggcs,
        W3[:_FEAT], W3[_FEAT:], b3[None, :], W4, b4[None, :])
    return h_new[None], x_new[None]
